# baseline (speedup 1.0000x reference)
import functools

import jax
import jax.numpy as jnp
from jax import lax
from jax.experimental import pallas as pl
from jax.experimental.pallas import tpu as pltpu
from jax.experimental.pallas import tpu_sc as plsc

_NC = 2
_NS = 16
_NW = _NC * _NS


def kernel(input_ids, wte):
    B, L = input_ids.shape
    V, D = wte.shape
    N = B * L
    idx_flat = input_ids.reshape(N).astype(jnp.int32)

    n_per_w = N // _NW
    CH = 128
    n_ch = n_per_w // CH

    mesh = plsc.VectorSubcoreMesh(core_axis_name="c", subcore_axis_name="s")

    @functools.partial(
        pl.kernel,
        mesh=mesh,
        out_type=jax.ShapeDtypeStruct((N, D), jnp.float32),
        compiler_params=pltpu.CompilerParams(use_tc_tiling_on_sc=False),
        scratch_types=[
            pltpu.VMEM((n_per_w,), jnp.int32),
            pltpu.VMEM((CH, D), jnp.float32),
            pltpu.SemaphoreType.DMA,
        ],
    )
    def emb(idx_hbm, table_hbm, out_hbm, idx_v, rows_v, sem):
        wid = lax.axis_index("s") * _NC + lax.axis_index("c")
        base = wid * n_per_w
        pltpu.sync_copy(idx_hbm.at[pl.ds(base, n_per_w)], idx_v)

        def body(g, carry):
            off = g * CH
            pltpu.async_copy(
                table_hbm.at[idx_v.at[pl.ds(off, CH)]], rows_v, sem
            ).wait()
            pltpu.sync_copy(rows_v, out_hbm.at[pl.ds(base + off, CH)])
            return carry

        lax.fori_loop(0, n_ch, body, 0)

    out = emb(idx_flat, wte)
    return out.reshape(B, L, D)


# speedup vs baseline: 1.0621x; 1.0621x over previous
import functools

import jax
import jax.numpy as jnp
from jax import lax
from jax.experimental import pallas as pl
from jax.experimental.pallas import tpu as pltpu
from jax.experimental.pallas import tpu_sc as plsc

_NC = 2
_NS = 16
_NW = _NC * _NS

_CH = 128
_G = 5
_CB = _CH * _G


def kernel(input_ids, wte):
    B, L = input_ids.shape
    V, D = wte.shape
    N = B * L
    idx_flat = input_ids.reshape(N).astype(jnp.int32)

    n_per_w = N // _NW
    n_blk = n_per_w // _CB
    assert n_per_w % _CB == 0 and n_blk % 2 == 0

    mesh = plsc.VectorSubcoreMesh(core_axis_name="c", subcore_axis_name="s")

    @functools.partial(
        pl.kernel,
        mesh=mesh,
        out_type=jax.ShapeDtypeStruct((N, D), jnp.float32),
        compiler_params=pltpu.CompilerParams(use_tc_tiling_on_sc=False),
        scratch_types=[
            pltpu.VMEM((n_per_w,), jnp.int32),
            pltpu.VMEM((_CB, D), jnp.float32),
            pltpu.VMEM((_CB, D), jnp.float32),
            pltpu.SemaphoreType.DMA,
            pltpu.SemaphoreType.DMA,
            pltpu.SemaphoreType.DMA,
            pltpu.SemaphoreType.DMA,
        ],
    )
    def emb(idx_hbm, table_hbm, out_hbm, idx_v, buf0, buf1, sg0, sg1, sw0, sw1):
        wid = lax.axis_index("s") * _NC + lax.axis_index("c")
        base = wid * n_per_w
        pltpu.sync_copy(idx_hbm.at[pl.ds(base, n_per_w)], idx_v)

        bufs = (buf0, buf1)
        sgs = (sg0, sg1)
        sws = (sw0, sw1)

        def fire_gathers(blk, buf, sem):
            off = blk * _CB
            for j in range(_G):
                pltpu.async_copy(
                    table_hbm.at[idx_v.at[pl.ds(off + j * _CH, _CH)]],
                    buf.at[pl.ds(j * _CH, _CH)],
                    sem,
                )

        def drain_gathers(buf, sem):
            pltpu.make_async_copy(table_hbm.at[pl.ds(0, _CB)], buf, sem).wait()

        def fire_writeout(blk, buf, sem):
            pltpu.async_copy(buf, out_hbm.at[pl.ds(base + blk * _CB, _CB)], sem)

        def drain_writeout(buf, sem):
            pltpu.make_async_copy(buf, out_hbm.at[pl.ds(base, _CB)], sem).wait()

        fire_gathers(0, buf0, sg0)

        def body(t, carry):
            for b in range(2):
                blk = 2 * t + b
                p, q = b % 2, (b + 1) % 2
                drain_gathers(bufs[p], sgs[p])

                @pl.when(jnp.logical_and(blk >= 1, blk + 1 < n_blk))
                def _():
                    drain_writeout(bufs[q], sws[q])

                @pl.when(blk + 1 < n_blk)
                def _():
                    fire_gathers(blk + 1, bufs[q], sgs[q])

                fire_writeout(blk, bufs[p], sws[p])
            return carry

        lax.fori_loop(0, n_blk // 2, body, 0)
        drain_writeout(bufs[0], sws[0])
        drain_writeout(bufs[1], sws[1])

    out = emb(idx_flat, wte)
    return out.reshape(B, L, D)
